# baseline (speedup 1.0000x reference)
import functools

import numpy as np
import jax
import jax.numpy as jnp
from jax import lax
from jax.experimental import pallas as pl
from jax.experimental.pallas import tpu as pltpu
from jax.experimental.pallas import tpu_sc as plsc

B, N, S, K = 8, 8192, 512, 32
SBLK = 128
R2 = np.float32(0.4 * 0.4)
COUT = 64

NW = 32
RPW = (B * S * K) // NW
CHUNK = 128
NCH = RPW // CHUNK
GRP = 4


def _fps_body(xs_ref, ys_ref, zs_ref, nx_ref, ny_ref, nz_ref, d_ref):
    xs = xs_ref[...]
    ys = ys_ref[...]
    zs = zs_ref[...]
    lane = lax.broadcasted_iota(jnp.int32, (B, N), 1)
    col = lax.broadcasted_iota(jnp.int32, (B, S), 1)
    d_ref[...] = jnp.full((B, N), 1e10, jnp.float32)

    def body(i, far):
        oh = lane == far
        cx = jnp.sum(jnp.where(oh, xs, 0.0), axis=1, keepdims=True)
        cy = jnp.sum(jnp.where(oh, ys, 0.0), axis=1, keepdims=True)
        cz = jnp.sum(jnp.where(oh, zs, 0.0), axis=1, keepdims=True)
        sel = col == i
        nx_ref[...] = jnp.where(sel, cx, nx_ref[...])
        ny_ref[...] = jnp.where(sel, cy, ny_ref[...])
        nz_ref[...] = jnp.where(sel, cz, nz_ref[...])
        dx = xs - cx
        dy = ys - cy
        dz = zs - cz
        d = dx * dx + dy * dy + dz * dz
        nd = jnp.minimum(d_ref[...], d)
        d_ref[...] = nd
        m = jnp.max(nd, axis=1, keepdims=True)
        return jnp.min(jnp.where(nd == m, lane, N), axis=1, keepdims=True)

    lax.fori_loop(0, S, body, jnp.zeros((B, 1), jnp.int32))


def _fps(xs, ys, zs):
    return pl.pallas_call(
        _fps_body,
        out_shape=[jax.ShapeDtypeStruct((B, S), jnp.float32)] * 3,
        scratch_shapes=[pltpu.VMEM((B, N), jnp.float32)],
    )(xs, ys, zs)


def _bq_body(ns_ref, xt_ref, out_ref, cnt_ref, carry_ref):
    b = pl.program_id(0)
    ns = ns_ref[0]
    xt = xt_ref[0]
    csq = jnp.sum(ns * ns, axis=1, keepdims=True)
    CB = 512
    r = lax.broadcasted_iota(jnp.int32, (CB, CB), 0)
    c = lax.broadcasted_iota(jnp.int32, (CB, CB), 1)
    tri = jnp.where(r <= c, 1.0, 0.0).astype(jnp.float32)
    dn = (((1,), (0,)), ((), ()))
    cnt_ref[...] = jnp.zeros((SBLK, K), jnp.float32)
    carry_ref[...] = jnp.zeros((SBLK, 1), jnp.float32)
    for ci in range(N // CB):
        @pl.when(jnp.min(carry_ref[...]) < np.float32(K))
        def _chunk():
            xt_c = xt[:, ci * CB:(ci + 1) * CB]
            cross = lax.dot_general(ns, xt_c, dn,
                                    preferred_element_type=jnp.float32)
            xsq = jnp.sum(xt_c * xt_c, axis=0, keepdims=True)
            sqr = (csq - 2.0 * cross) + xsq
            maskf = jnp.where(sqr <= R2, 1.0, 0.0)
            lc = lax.dot_general(maskf, tri, dn,
                                 preferred_element_type=jnp.float32)
            lc = lc + carry_ref[...]
            cols = [jnp.sum(jnp.where(lc <= np.float32(kb), 1.0, 0.0),
                            axis=1, keepdims=True) for kb in range(K)]
            cnt_ref[...] += jnp.concatenate(cols, axis=1)
            carry_ref[...] = lc[:, CB - 1:CB]
    cnt = cnt_ref[...].astype(jnp.int32)
    first = cnt[:, 0:1]
    gid = jnp.where(cnt == N, first, cnt)
    gid = jnp.minimum(gid, N - 1)
    out_ref[0] = gid + b * N


def _ballq(new_s, xt):
    return pl.pallas_call(
        _bq_body,
        grid=(B, S // SBLK),
        in_specs=[
            pl.BlockSpec((1, SBLK, 3), lambda b, j: (b, j, 0)),
            pl.BlockSpec((1, 3, N), lambda b, j: (b, 0, 0)),
        ],
        out_specs=pl.BlockSpec((1, SBLK, K), lambda b, j: (b, j, 0)),
        out_shape=jax.ShapeDtypeStruct((B, S, K), jnp.int32),
        scratch_shapes=[pltpu.VMEM((SBLK, K), jnp.float32),
                        pltpu.VMEM((SBLK, 1), jnp.float32)],
    )(new_s, xt)


def _mlp_body(x_ref, *refs):
    o_ref = refs[-1]
    dn = (((1,), (0,)), ((), ()))
    h = x_ref[...]
    for li in range(3):
        wt, bb, g, be, m, v = (r[...] for r in refs[li * 6:(li + 1) * 6])
        y = lax.dot_general(h, wt, dn, preferred_element_type=jnp.float32) + bb
        y = (y - m) / jnp.sqrt(v + 1e-5) * g + be
        h = jnp.maximum(y, 0.0)
    o_ref[...] = h


def _mlp(x, params):
    rows = B * N
    rblk = 4096
    full = lambda i: (0, 0)
    specs = [pl.BlockSpec((rblk, 16), lambda i: (i, 0))]
    flat = []
    for (wt, bb, g, be, m, v) in params:
        specs.append(pl.BlockSpec(wt.shape, full))
        flat.append(wt)
        for p in (bb, g, be, m, v):
            specs.append(pl.BlockSpec((1, p.shape[1]), full))
            flat.append(p)
    return pl.pallas_call(
        _mlp_body,
        grid=(rows // rblk,),
        in_specs=specs,
        out_specs=pl.BlockSpec((rblk, COUT), lambda i: (i, 0)),
        out_shape=jax.ShapeDtypeStruct((rows, COUT), jnp.float32),
    )(x, *flat)


def _sc_gather(feats, idx2d):
    mesh = plsc.VectorSubcoreMesh(core_axis_name="c", subcore_axis_name="s")

    @functools.partial(
        pl.kernel,
        out_type=jax.ShapeDtypeStruct((B * S * K, COUT), jnp.float32),
        mesh=mesh,
        compiler_params=pltpu.CompilerParams(use_tc_tiling_on_sc=False),
        scratch_types=[
            pltpu.VMEM((NCH, CHUNK), jnp.int32),
            pltpu.VMEM((GRP * CHUNK, COUT), jnp.float32),
            pltpu.SemaphoreType.DMA,
        ],
    )
    def k(feats_hbm, idx_hbm, out_hbm, idx_v, rows_v, sem):
        wid = lax.axis_index("s") * 2 + lax.axis_index("c")
        pltpu.sync_copy(idx_hbm.at[pl.ds(wid * NCH, NCH)], idx_v)
        for g in range(NCH // GRP):
            cps = []
            for t in range(GRP):
                j = g * GRP + t
                cps.append(pltpu.async_copy(
                    feats_hbm.at[idx_v.at[j]],
                    rows_v.at[pl.ds(t * CHUNK, CHUNK)], sem))
            for c in cps:
                c.wait()
            pltpu.sync_copy(
                rows_v,
                out_hbm.at[pl.ds(wid * RPW + g * GRP * CHUNK, GRP * CHUNK)])

    return k(feats, idx2d)


def kernel(xyz, points, W0, b0, gamma0, beta0, mean0, var0,
           W1, b1, gamma1, beta1, mean1, var1,
           W2, b2, gamma2, beta2, mean2, var2):
    nx, ny, nz = _fps(xyz[:, :, 0], xyz[:, :, 1], xyz[:, :, 2])
    new_s = jnp.stack([nx, ny, nz], axis=-1)
    xt = jnp.transpose(xyz, (0, 2, 1))
    fidx = _ballq(new_s, xt)

    row = lambda p: p[None, :]
    params = [
        (W0.T, row(b0), row(gamma0), row(beta0), row(mean0), row(var0)),
        (W1.T, row(b1), row(gamma1), row(beta1), row(mean1), row(var1)),
        (W2.T, row(b2), row(gamma2), row(beta2), row(mean2), row(var2)),
    ]
    feats = _mlp(points.reshape(B * N, 16), params)
    idx2d = jnp.minimum(fidx.reshape(-1), B * N - 1).reshape(-1, CHUNK)
    out = _sc_gather(feats, idx2d)
    return out.reshape(B, S, K, COUT)


# speedup vs baseline: 1.4996x; 1.4996x over previous
import functools

import numpy as np
import jax
import jax.numpy as jnp
from jax import lax
from jax.experimental import pallas as pl
from jax.experimental.pallas import tpu as pltpu
from jax.experimental.pallas import tpu_sc as plsc

B, N, S, K = 8, 8192, 512, 32
SBLK = 128
R2 = np.float32(0.4 * 0.4)
COUT = 64

NW = 32
RPW = (B * S * K) // NW
CHUNK = 128
NCH = RPW // CHUNK
GRP = 4


def _fps_body(xs_ref, ys_ref, zs_ref, nx_ref, ny_ref, nz_ref, d_ref):
    xs = xs_ref[...]
    ys = ys_ref[...]
    zs = zs_ref[...]
    lane = lax.broadcasted_iota(jnp.int32, (B, N), 1)
    col = lax.broadcasted_iota(jnp.int32, (B, S), 1)
    d_ref[...] = jnp.full((B, N), 1e10, jnp.float32)

    def body(i, far):
        oh = lane == far
        cx = jnp.sum(jnp.where(oh, xs, 0.0), axis=1, keepdims=True)
        cy = jnp.sum(jnp.where(oh, ys, 0.0), axis=1, keepdims=True)
        cz = jnp.sum(jnp.where(oh, zs, 0.0), axis=1, keepdims=True)
        sel = col == i
        nx_ref[...] = jnp.where(sel, cx, nx_ref[...])
        ny_ref[...] = jnp.where(sel, cy, ny_ref[...])
        nz_ref[...] = jnp.where(sel, cz, nz_ref[...])
        dx = xs - cx
        dy = ys - cy
        dz = zs - cz
        d = dx * dx + dy * dy + dz * dz
        nd = jnp.minimum(d_ref[...], d)
        d_ref[...] = nd
        m = jnp.max(nd, axis=1, keepdims=True)
        return jnp.min(jnp.where(nd == m, lane, N), axis=1, keepdims=True)

    lax.fori_loop(0, S, body, jnp.zeros((B, 1), jnp.int32))


def _fps(xs, ys, zs):
    return pl.pallas_call(
        _fps_body,
        out_shape=[jax.ShapeDtypeStruct((B, S), jnp.float32)] * 3,
        scratch_shapes=[pltpu.VMEM((B, N), jnp.float32)],
    )(xs, ys, zs)


def _bq_body(ns_ref, xt_ref, out_ref):
    b = pl.program_id(0)
    ns = ns_ref[0]
    xt = xt_ref[0]
    xsq = jnp.sum(xt * xt, axis=0, keepdims=True)
    csq = jnp.sum(ns * ns, axis=1, keepdims=True)
    dn = (((1,), (0,)), ((), ()))
    cross = lax.dot_general(ns, xt, dn,
                            preferred_element_type=jnp.float32)
    sqr = (csq - 2.0 * cross) + xsq
    maskf = jnp.where(sqr <= R2, 1.0, 0.0)
    CB = 512
    r = lax.broadcasted_iota(jnp.int32, (CB, CB), 0)
    c = lax.broadcasted_iota(jnp.int32, (CB, CB), 1)
    tri = jnp.where(r <= c, 1.0, 0.0).astype(jnp.float32)
    carry = jnp.zeros((SBLK, 1), jnp.float32)
    parts = []
    for ci in range(N // CB):
        blk = maskf[:, ci * CB:(ci + 1) * CB]
        lc = lax.dot_general(blk, tri, dn,
                             preferred_element_type=jnp.float32) + carry
        parts.append(lc)
        carry = lc[:, CB - 1:CB]
    cm = jnp.concatenate(parts, axis=1)
    cmb = cm.astype(jnp.bfloat16)
    one_b = jnp.ones((SBLK, 256), jnp.bfloat16)
    zero_b = jnp.zeros((SBLK, 256), jnp.bfloat16)
    cols = []
    for k in range(K):
        acc = zero_b
        for p in range(N // 256):
            acc = acc + jnp.where(cmb[:, p * 256:(p + 1) * 256]
                                  <= jnp.bfloat16(k), one_b, zero_b)
        cols.append(jnp.sum(acc.astype(jnp.float32), axis=1, keepdims=True))
    cnt = jnp.concatenate(cols, axis=1).astype(jnp.int32)
    first = cnt[:, 0:1]
    gid = jnp.where(cnt == N, first, cnt)
    gid = jnp.minimum(gid, N - 1)
    out_ref[0] = gid + b * N


def _ballq(new_s, xt):
    return pl.pallas_call(
        _bq_body,
        grid=(B, S // SBLK),
        in_specs=[
            pl.BlockSpec((1, SBLK, 3), lambda b, j: (b, j, 0)),
            pl.BlockSpec((1, 3, N), lambda b, j: (b, 0, 0)),
        ],
        out_specs=pl.BlockSpec((1, SBLK, K), lambda b, j: (b, j, 0)),
        out_shape=jax.ShapeDtypeStruct((B, S, K), jnp.int32),
    )(new_s, xt)


def _mlp_body(x_ref, *refs):
    o_ref = refs[-1]
    dn = (((1,), (0,)), ((), ()))
    h = x_ref[...]
    for li in range(3):
        wt, bb, g, be, m, v = (r[...] for r in refs[li * 6:(li + 1) * 6])
        y = lax.dot_general(h, wt, dn, preferred_element_type=jnp.float32) + bb
        y = (y - m) / jnp.sqrt(v + 1e-5) * g + be
        h = jnp.maximum(y, 0.0)
    o_ref[...] = h


def _mlp(x, params):
    rows = B * N
    rblk = 4096
    full = lambda i: (0, 0)
    specs = [pl.BlockSpec((rblk, 16), lambda i: (i, 0))]
    flat = []
    for (wt, bb, g, be, m, v) in params:
        specs.append(pl.BlockSpec(wt.shape, full))
        flat.append(wt)
        for p in (bb, g, be, m, v):
            specs.append(pl.BlockSpec((1, p.shape[1]), full))
            flat.append(p)
    return pl.pallas_call(
        _mlp_body,
        grid=(rows // rblk,),
        in_specs=specs,
        out_specs=pl.BlockSpec((rblk, COUT), lambda i: (i, 0)),
        out_shape=jax.ShapeDtypeStruct((rows, COUT), jnp.float32),
    )(x, *flat)


def _sc_gather(feats, idx2d):
    mesh = plsc.VectorSubcoreMesh(core_axis_name="c", subcore_axis_name="s")

    @functools.partial(
        pl.kernel,
        out_type=jax.ShapeDtypeStruct((B * S * K, COUT), jnp.float32),
        mesh=mesh,
        compiler_params=pltpu.CompilerParams(use_tc_tiling_on_sc=False),
        scratch_types=[
            pltpu.VMEM((NCH, CHUNK), jnp.int32),
            pltpu.VMEM((GRP * CHUNK, COUT), jnp.float32),
            pltpu.SemaphoreType.DMA,
        ],
    )
    def k(feats_hbm, idx_hbm, out_hbm, idx_v, rows_v, sem):
        wid = lax.axis_index("s") * 2 + lax.axis_index("c")
        pltpu.sync_copy(idx_hbm.at[pl.ds(wid * NCH, NCH)], idx_v)
        for g in range(NCH // GRP):
            cps = []
            for t in range(GRP):
                j = g * GRP + t
                cps.append(pltpu.async_copy(
                    feats_hbm.at[idx_v.at[j]],
                    rows_v.at[pl.ds(t * CHUNK, CHUNK)], sem))
            for c in cps:
                c.wait()
            pltpu.sync_copy(
                rows_v,
                out_hbm.at[pl.ds(wid * RPW + g * GRP * CHUNK, GRP * CHUNK)])

    return k(feats, idx2d)


def kernel(xyz, points, W0, b0, gamma0, beta0, mean0, var0,
           W1, b1, gamma1, beta1, mean1, var1,
           W2, b2, gamma2, beta2, mean2, var2):
    nx, ny, nz = _fps(xyz[:, :, 0], xyz[:, :, 1], xyz[:, :, 2])
    new_s = jnp.stack([nx, ny, nz], axis=-1)
    xt = jnp.transpose(xyz, (0, 2, 1))
    fidx = _ballq(new_s, xt)

    row = lambda p: p[None, :]
    params = [
        (W0.T, row(b0), row(gamma0), row(beta0), row(mean0), row(var0)),
        (W1.T, row(b1), row(gamma1), row(beta1), row(mean1), row(var1)),
        (W2.T, row(b2), row(gamma2), row(beta2), row(mean2), row(var2)),
    ]
    feats = _mlp(points.reshape(B * N, 16), params)
    idx2d = jnp.minimum(fidx.reshape(-1), B * N - 1).reshape(-1, CHUNK)
    out = _sc_gather(feats, idx2d)
    return out.reshape(B, S, K, COUT)


# speedup vs baseline: 1.5091x; 1.0063x over previous
import functools

import numpy as np
import jax
import jax.numpy as jnp
from jax import lax
from jax.experimental import pallas as pl
from jax.experimental.pallas import tpu as pltpu
from jax.experimental.pallas import tpu_sc as plsc

B, N, S, K = 8, 8192, 512, 32
SBLK = 128
R2 = np.float32(0.4 * 0.4)
COUT = 64

NW = 32
RPW = (B * S * K) // NW
CHUNK = 128
NCH = RPW // CHUNK
GRP = 4


def _fps_body(xs_ref, ys_ref, zs_ref, nx_ref, ny_ref, nz_ref, d_ref):
    xs = xs_ref[...]
    ys = ys_ref[...]
    zs = zs_ref[...]
    lane = lax.broadcasted_iota(jnp.int32, (B, N), 1)
    col = lax.broadcasted_iota(jnp.int32, (B, S), 1)
    d_ref[...] = jnp.full((B, N), 1e10, jnp.float32)

    def body(i, far):
        oh = lane == far
        cx = jnp.sum(jnp.where(oh, xs, 0.0), axis=1, keepdims=True)
        cy = jnp.sum(jnp.where(oh, ys, 0.0), axis=1, keepdims=True)
        cz = jnp.sum(jnp.where(oh, zs, 0.0), axis=1, keepdims=True)
        sel = col == i
        nx_ref[...] = jnp.where(sel, cx, nx_ref[...])
        ny_ref[...] = jnp.where(sel, cy, ny_ref[...])
        nz_ref[...] = jnp.where(sel, cz, nz_ref[...])
        dx = xs - cx
        dy = ys - cy
        dz = zs - cz
        d = dx * dx + dy * dy + dz * dz
        nd = jnp.minimum(d_ref[...], d)
        d_ref[...] = nd
        m = jnp.max(nd, axis=1, keepdims=True)
        return jnp.min(jnp.where(nd == m, lane, N), axis=1, keepdims=True)

    lax.fori_loop(0, S, body, jnp.zeros((B, 1), jnp.int32))


def _fps(xs, ys, zs):
    return pl.pallas_call(
        _fps_body,
        out_shape=[jax.ShapeDtypeStruct((B, S), jnp.float32)] * 3,
        scratch_shapes=[pltpu.VMEM((B, N), jnp.float32)],
    )(xs, ys, zs)


def _bq_body(nx_ref, ny_ref, nz_ref, xs_ref, ys_ref, zs_ref, out_ref):
    b = pl.program_id(0)
    ns = jnp.transpose(
        jnp.concatenate([nx_ref[pl.ds(b, 1), :], ny_ref[pl.ds(b, 1), :],
                         nz_ref[pl.ds(b, 1), :]], axis=0),
        (1, 0))
    xt = jnp.concatenate([xs_ref[pl.ds(b, 1), :], ys_ref[pl.ds(b, 1), :],
                          zs_ref[pl.ds(b, 1), :]], axis=0)
    xsq = jnp.sum(xt * xt, axis=0, keepdims=True)
    csq = jnp.sum(ns * ns, axis=1, keepdims=True)
    dn = (((1,), (0,)), ((), ()))
    cross = lax.dot_general(ns, xt, dn,
                            preferred_element_type=jnp.float32)
    sqr = (csq - 2.0 * cross) + xsq
    maskf = jnp.where(sqr <= R2, 1.0, 0.0)
    CB = 512
    r = lax.broadcasted_iota(jnp.int32, (CB, CB), 0)
    c = lax.broadcasted_iota(jnp.int32, (CB, CB), 1)
    tri = jnp.where(r <= c, 1.0, 0.0).astype(jnp.float32)
    carry = jnp.zeros((SBLK, 1), jnp.float32)
    parts = []
    for ci in range(N // CB):
        blk = maskf[:, ci * CB:(ci + 1) * CB]
        lc = lax.dot_general(blk, tri, dn,
                             preferred_element_type=jnp.float32) + carry
        parts.append(lc)
        carry = lc[:, CB - 1:CB]
    cm = jnp.concatenate(parts, axis=1)
    cmb = cm.astype(jnp.bfloat16)
    one_b = jnp.ones((SBLK, 256), jnp.bfloat16)
    zero_b = jnp.zeros((SBLK, 256), jnp.bfloat16)
    cols = []
    for k in range(K):
        acc = zero_b
        for p in range(N // 256):
            acc = acc + jnp.where(cmb[:, p * 256:(p + 1) * 256]
                                  <= jnp.bfloat16(k), one_b, zero_b)
        cols.append(jnp.sum(acc.astype(jnp.float32), axis=1, keepdims=True))
    cnt = jnp.concatenate(cols, axis=1).astype(jnp.int32)
    first = cnt[:, 0:1]
    gid = jnp.where(cnt == N, first, cnt)
    gid = jnp.minimum(gid, N - 1)
    out_ref[0] = gid + b * N


def _ballq(nx, ny, nz, xs, ys, zs):
    nspec = pl.BlockSpec((B, SBLK), lambda b, j: (0, j))
    xspec = pl.BlockSpec((B, N), lambda b, j: (0, 0))
    return pl.pallas_call(
        _bq_body,
        grid=(B, S // SBLK),
        in_specs=[nspec, nspec, nspec, xspec, xspec, xspec],
        out_specs=pl.BlockSpec((1, SBLK, K), lambda b, j: (b, j, 0)),
        out_shape=jax.ShapeDtypeStruct((B, S, K), jnp.int32),
    )(nx, ny, nz, xs, ys, zs)


def _mlp_body(x_ref, *refs):
    o_ref = refs[-1]
    dn = (((1,), (0,)), ((), ()))
    h = x_ref[...]
    for li in range(3):
        wt, bb, g, be, m, v = (r[...] for r in refs[li * 6:(li + 1) * 6])
        y = lax.dot_general(h, wt, dn, preferred_element_type=jnp.float32) + bb
        y = (y - m) / jnp.sqrt(v + 1e-5) * g + be
        h = jnp.maximum(y, 0.0)
    o_ref[...] = h


def _mlp(x, params):
    rows = B * N
    rblk = 4096
    full = lambda i: (0, 0)
    specs = [pl.BlockSpec((rblk, 16), lambda i: (i, 0))]
    flat = []
    for (wt, bb, g, be, m, v) in params:
        specs.append(pl.BlockSpec(wt.shape, full))
        flat.append(wt)
        for p in (bb, g, be, m, v):
            specs.append(pl.BlockSpec((1, p.shape[1]), full))
            flat.append(p)
    return pl.pallas_call(
        _mlp_body,
        grid=(rows // rblk,),
        in_specs=specs,
        out_specs=pl.BlockSpec((rblk, COUT), lambda i: (i, 0)),
        out_shape=jax.ShapeDtypeStruct((rows, COUT), jnp.float32),
    )(x, *flat)


def _sc_gather(feats, idx2d):
    mesh = plsc.VectorSubcoreMesh(core_axis_name="c", subcore_axis_name="s")

    @functools.partial(
        pl.kernel,
        out_type=jax.ShapeDtypeStruct((B * S * K, COUT), jnp.float32),
        mesh=mesh,
        compiler_params=pltpu.CompilerParams(use_tc_tiling_on_sc=False),
        scratch_types=[
            pltpu.VMEM((NCH, CHUNK), jnp.int32),
            pltpu.VMEM((GRP * CHUNK, COUT), jnp.float32),
            pltpu.SemaphoreType.DMA,
        ],
    )
    def k(feats_hbm, idx_hbm, out_hbm, idx_v, rows_v, sem):
        wid = lax.axis_index("s") * 2 + lax.axis_index("c")
        pltpu.sync_copy(idx_hbm.at[pl.ds(wid * NCH, NCH)], idx_v)
        for g in range(NCH // GRP):
            cps = []
            for t in range(GRP):
                j = g * GRP + t
                cps.append(pltpu.async_copy(
                    feats_hbm.at[idx_v.at[j]],
                    rows_v.at[pl.ds(t * CHUNK, CHUNK)], sem))
            for c in cps:
                c.wait()
            pltpu.sync_copy(
                rows_v,
                out_hbm.at[pl.ds(wid * RPW + g * GRP * CHUNK, GRP * CHUNK)])

    return k(feats, idx2d)


def kernel(xyz, points, W0, b0, gamma0, beta0, mean0, var0,
           W1, b1, gamma1, beta1, mean1, var1,
           W2, b2, gamma2, beta2, mean2, var2):
    xs, ys, zs = xyz[:, :, 0], xyz[:, :, 1], xyz[:, :, 2]
    nx, ny, nz = _fps(xs, ys, zs)
    fidx = _ballq(nx, ny, nz, xs, ys, zs)

    row = lambda p: p[None, :]
    params = [
        (W0.T, row(b0), row(gamma0), row(beta0), row(mean0), row(var0)),
        (W1.T, row(b1), row(gamma1), row(beta1), row(mean1), row(var1)),
        (W2.T, row(b2), row(gamma2), row(beta2), row(mean2), row(var2)),
    ]
    feats = _mlp(points.reshape(B * N, 16), params)
    idx2d = jnp.minimum(fidx.reshape(-1), B * N - 1).reshape(-1, CHUNK)
    out = _sc_gather(feats, idx2d)
    return out.reshape(B, S, K, COUT)


# speedup vs baseline: 1.5096x; 1.0004x over previous
import functools

import numpy as np
import jax
import jax.numpy as jnp
from jax import lax
from jax.experimental import pallas as pl
from jax.experimental.pallas import tpu as pltpu
from jax.experimental.pallas import tpu_sc as plsc

B, N, S, K = 8, 8192, 512, 32
SBLK = 128
R2 = np.float32(0.4 * 0.4)
COUT = 64

NW = 32
RPW = (B * S * K) // NW
CHUNK = 128
NCH = RPW // CHUNK
GRP = 4


def _fps_body(xs_ref, ys_ref, zs_ref, nx_ref, ny_ref, nz_ref, d_ref):
    xs = xs_ref[...]
    ys = ys_ref[...]
    zs = zs_ref[...]
    lane = lax.broadcasted_iota(jnp.int32, (B, N), 1)
    col = lax.broadcasted_iota(jnp.int32, (B, S), 1)
    d_ref[...] = jnp.full((B, N), 1e10, jnp.float32)

    def body(i, far):
        oh = lane == far
        cx = jnp.sum(jnp.where(oh, xs, 0.0), axis=1, keepdims=True)
        cy = jnp.sum(jnp.where(oh, ys, 0.0), axis=1, keepdims=True)
        cz = jnp.sum(jnp.where(oh, zs, 0.0), axis=1, keepdims=True)
        sel = col == i
        nx_ref[...] = jnp.where(sel, cx, nx_ref[...])
        ny_ref[...] = jnp.where(sel, cy, ny_ref[...])
        nz_ref[...] = jnp.where(sel, cz, nz_ref[...])
        dx = xs - cx
        dy = ys - cy
        dz = zs - cz
        d = dx * dx + dy * dy + dz * dz
        nd = jnp.minimum(d_ref[...], d)
        d_ref[...] = nd
        m = jnp.max(nd, axis=1, keepdims=True)
        return jnp.min(jnp.where(nd == m, lane, N), axis=1, keepdims=True)

    lax.fori_loop(0, S, body, jnp.zeros((B, 1), jnp.int32))


def _fps(xs, ys, zs):
    return pl.pallas_call(
        _fps_body,
        out_shape=[jax.ShapeDtypeStruct((B, S), jnp.float32)] * 3,
        scratch_shapes=[pltpu.VMEM((B, N), jnp.float32)],
    )(xs, ys, zs)


def _bq_body(nx_ref, ny_ref, nz_ref, xs_ref, ys_ref, zs_ref, out_ref):
    b = pl.program_id(0)
    ns = jnp.transpose(
        jnp.concatenate([nx_ref[pl.ds(b, 1), :], ny_ref[pl.ds(b, 1), :],
                         nz_ref[pl.ds(b, 1), :]], axis=0),
        (1, 0))
    xt = jnp.concatenate([xs_ref[pl.ds(b, 1), :], ys_ref[pl.ds(b, 1), :],
                          zs_ref[pl.ds(b, 1), :]], axis=0)
    xsq = jnp.sum(xt * xt, axis=0, keepdims=True)
    csq = jnp.sum(ns * ns, axis=1, keepdims=True)
    dn = (((1,), (0,)), ((), ()))
    cross = lax.dot_general(ns, xt, dn,
                            preferred_element_type=jnp.float32)
    sqr = (csq - 2.0 * cross) + xsq
    maskf = jnp.where(sqr <= R2, 1.0, 0.0)
    CB = 512
    r = lax.broadcasted_iota(jnp.int32, (CB, CB), 0)
    c = lax.broadcasted_iota(jnp.int32, (CB, CB), 1)
    tri = jnp.where(r <= c, 1.0, 0.0).astype(jnp.float32)
    carry = jnp.zeros((SBLK, 1), jnp.float32)
    parts = []
    for ci in range(N // CB):
        blk = maskf[:, ci * CB:(ci + 1) * CB]
        lc = lax.dot_general(blk, tri, dn,
                             preferred_element_type=jnp.float32) + carry
        parts.append(lc)
        carry = lc[:, CB - 1:CB]
    cm = jnp.concatenate(parts, axis=1)
    cmb = cm.astype(jnp.bfloat16)
    one_b = jnp.ones((SBLK, 256), jnp.bfloat16)
    zero_b = jnp.zeros((SBLK, 256), jnp.bfloat16)
    cols = []
    for k in range(K):
        acc = zero_b
        for p in range(N // 256):
            acc = acc + jnp.where(cmb[:, p * 256:(p + 1) * 256]
                                  <= jnp.bfloat16(k), one_b, zero_b)
        cols.append(jnp.sum(acc.astype(jnp.float32), axis=1, keepdims=True))
    cnt = jnp.concatenate(cols, axis=1).astype(jnp.int32)
    first = cnt[:, 0:1]
    gid = jnp.where(cnt == N, first, cnt)
    gid = jnp.minimum(gid, N - 1)
    out_ref[0] = gid + b * N


def _ballq(nx, ny, nz, xs, ys, zs):
    nspec = pl.BlockSpec((B, SBLK), lambda b, j: (0, j))
    xspec = pl.BlockSpec((B, N), lambda b, j: (0, 0))
    return pl.pallas_call(
        _bq_body,
        grid=(B, S // SBLK),
        in_specs=[nspec, nspec, nspec, xspec, xspec, xspec],
        out_specs=pl.BlockSpec((1, SBLK, K), lambda b, j: (b, j, 0)),
        out_shape=jax.ShapeDtypeStruct((B, S, K), jnp.int32),
    )(nx, ny, nz, xs, ys, zs)


def _mlp_body(x_ref, *refs):
    o_ref = refs[-1]
    dn = (((1,), (0,)), ((), ()))
    h = x_ref[...]
    for li in range(3):
        wt, bb, g, be, m, v = (r[...] for r in refs[li * 6:(li + 1) * 6])
        y = lax.dot_general(h, wt, dn, preferred_element_type=jnp.float32) + bb
        y = (y - m) / jnp.sqrt(v + 1e-5) * g + be
        h = jnp.maximum(y, 0.0)
    o_ref[...] = h


def _mlp(x, params):
    rows = B * N
    rblk = 4096
    full = lambda i: (0, 0)
    specs = [pl.BlockSpec((rblk, 16), lambda i: (i, 0))]
    flat = []
    for (wt, bb, g, be, m, v) in params:
        specs.append(pl.BlockSpec(wt.shape, full))
        flat.append(wt)
        for p in (bb, g, be, m, v):
            specs.append(pl.BlockSpec((1, p.shape[1]), full))
            flat.append(p)
    return pl.pallas_call(
        _mlp_body,
        grid=(rows // rblk,),
        in_specs=specs,
        out_specs=pl.BlockSpec((rblk, COUT), lambda i: (i, 0)),
        out_shape=jax.ShapeDtypeStruct((rows, COUT), jnp.float32),
    )(x, *flat)


def _sc_gather(feats, idx2d):
    mesh = plsc.VectorSubcoreMesh(core_axis_name="c", subcore_axis_name="s")

    @functools.partial(
        pl.kernel,
        out_type=jax.ShapeDtypeStruct((B * S * K, COUT), jnp.float32),
        mesh=mesh,
        compiler_params=pltpu.CompilerParams(use_tc_tiling_on_sc=False),
        scratch_types=[
            pltpu.VMEM((NCH, CHUNK), jnp.int32),
            pltpu.VMEM((GRP * CHUNK, COUT), jnp.float32),
            pltpu.SemaphoreType.DMA,
        ],
    )
    def k(feats_hbm, idx_hbm, out_hbm, idx_v, rows_v, sem):
        wid = lax.axis_index("s") * 2 + lax.axis_index("c")
        pltpu.sync_copy(idx_hbm.at[pl.ds(wid * NCH, NCH)], idx_v)
        for g in range(NCH // GRP):
            cps = []
            for t in range(GRP):
                j = g * GRP + t
                cps.append(pltpu.async_copy(
                    feats_hbm.at[idx_v.at[j]],
                    rows_v.at[pl.ds(t * CHUNK, CHUNK)], sem))
            for c in cps:
                c.wait()
            pltpu.sync_copy(
                rows_v,
                out_hbm.at[pl.ds(wid * RPW + g * GRP * CHUNK, GRP * CHUNK)])

    return k(feats, idx2d)


def kernel(xyz, points, W0, b0, gamma0, beta0, mean0, var0,
           W1, b1, gamma1, beta1, mean1, var1,
           W2, b2, gamma2, beta2, mean2, var2):
    xs, ys, zs = xyz[:, :, 0], xyz[:, :, 1], xyz[:, :, 2]
    nx, ny, nz = _fps(xs, ys, zs)
    fidx = _ballq(nx, ny, nz, xs, ys, zs)

    row = lambda p: p[None, :]
    params = [
        (W0.T, row(b0), row(gamma0), row(beta0), row(mean0), row(var0)),
        (W1.T, row(b1), row(gamma1), row(beta1), row(mean1), row(var1)),
        (W2.T, row(b2), row(gamma2), row(beta2), row(mean2), row(var2)),
    ]
    feats = _mlp(points.reshape(B * N, 16), params)
    idx2d = fidx.reshape(-1, CHUNK)
    out = _sc_gather(feats, idx2d)
    return out.reshape(B, S, K, COUT)
